# Initial kernel scaffold; baseline (speedup 1.0000x reference)
import jax
import jax.numpy as jnp
from jax.experimental import pallas as pl


def kernel(x):
    raise NotImplementedError("write your pallas kernel here")


# speedup vs baseline: 2.5889x; 2.5889x over previous
import functools

import jax
import jax.numpy as jnp
from jax import lax
from jax.experimental import pallas as pl
from jax.experimental.pallas import tpu as pltpu
from jax.experimental.pallas import tpu_sc as plsc

K = 64
ROWS = 128
N = 32768
L = 16
NV = N // L
NWORKERS = 32
RPW = ROWS // NWORKERS
CAP = 2048
NEG_INF_KEY = -(2 ** 31)


def _topk_body(x_hbm, o_hbm, row_v, hist_v, ckey_v, cidx_v):
    wid = lax.axis_index("s") * 2 + lax.axis_index("c")
    iota = lax.iota(jnp.int32, L)
    ones = jnp.ones((L,), jnp.int32)
    zeros_f = jnp.zeros((L,), jnp.float32)

    for r in range(RPW):
        row = wid * RPW + r
        base = row * N
        pltpu.sync_copy(x_hbm.at[pl.ds(base, N)], row_v)

        def zb(j, carry):
            hist_v[pl.ds(j * L, L)] = jnp.zeros((L,), jnp.int32)
            return carry

        lax.fori_loop(0, 4096 // L, zb, 0)

        def hist_body(i, carry):
            v = row_v[pl.ds(i * L, L)]
            u = lax.bitcast_convert_type(v, jnp.int32)
            key = u ^ (lax.shift_right_arithmetic(u, 31) & 0x7FFFFFFF)
            b = lax.shift_right_arithmetic(key, 20) + 2048
            plsc.addupdate_scatter(hist_v, [b], ones)
            return carry

        lax.fori_loop(0, NV, hist_body, 0)

        def scan_cond(c):
            return jnp.logical_not(c[2])

        def scan_body(c):
            j, cum, found, b1, cnt_above = c
            hv = hist_v[pl.ds(j * L, L)]
            s = jnp.sum(hv)
            found_here = (cum + s) >= K
            pref = plsc.cumsum(hv)
            suf_in = s - pref + hv
            cross = (cum + suf_in) >= K
            npos = jnp.sum(cross.astype(jnp.int32))
            i_star = npos - 1
            lane_sel = iota == i_star
            suf_ex = s - pref
            ca_here = cum + jnp.sum(jnp.where(lane_sel, suf_ex, 0))
            b1_here = j * L + i_star
            b1_n = jnp.where(found_here, b1_here, b1)
            ca_n = jnp.where(found_here, ca_here, cnt_above)
            cum_n = jnp.where(found_here, cum, cum + s)
            return (j - 1, cum_n, found | found_here, b1_n, ca_n)

        init = (jnp.int32(4096 // L - 1), jnp.int32(0), False,
                jnp.int32(0), jnp.int32(0))
        _, _, _, b1, cnt_above = lax.while_loop(scan_cond, scan_body, init)

        k1 = K - cnt_above
        lo_edge = lax.shift_left(b1 - 2048, 20)
        hi_m1 = lax.shift_left(b1 - 2047, 20) - 1

        def p2_body(i, coff):
            v = row_v[pl.ds(i * L, L)]
            u = lax.bitcast_convert_type(v, jnp.int32)
            key = u ^ (lax.shift_right_arithmetic(u, 31) & 0x7FFFFFFF)
            in_top = key > hi_m1
            row_v[pl.ds(i * L, L)] = jnp.where(
                in_top, jnp.maximum(v, 0.0), zeros_f)
            in_b = (key >= lo_edge) & jnp.logical_not(in_top)
            cnt = jnp.sum(in_b.astype(jnp.int32))

            @pl.when(cnt > 0)
            def _():
                plsc.store_compressed(
                    ckey_v.at[pl.ds(coff, L)], key, mask=in_b)
                plsc.store_compressed(
                    cidx_v.at[pl.ds(coff, L)], iota + i * L, mask=in_b)

            return jnp.minimum(coff + cnt, CAP - L)

        c = lax.fori_loop(0, NV, p2_body, jnp.int32(0))

        ckey_v[pl.ds(c, L)] = jnp.full((L,), NEG_INF_KEY, jnp.int32)
        nv = (c + L - 1) // L

        def bs_body(it, t):
            cand = t + lax.shift_left(1, 19 - it)

            def cnt_body(j, acc):
                kv = ckey_v[pl.ds(j * L, L)]
                return acc + jnp.sum((kv >= cand).astype(jnp.int32))

            cnt = lax.fori_loop(0, nv, cnt_body, jnp.int32(0))
            return jnp.where(cnt >= k1, cand, t)

        t = lax.fori_loop(0, 20, bs_body, lo_edge)

        def cntgt_body(j, acc):
            kv = ckey_v[pl.ds(j * L, L)]
            return acc + jnp.sum((kv > t).astype(jnp.int32))

        cnt_gt = lax.fori_loop(0, nv, cntgt_body, jnp.int32(0))

        def sc_body(j, ties_left):
            kv = ckey_v[pl.ds(j * L, L)]
            iv = cidx_v[pl.ds(j * L, L)]
            gt = kv > t
            eq = kv == t
            pr = plsc.cumsum(eq.astype(jnp.int32))
            take = eq & (pr <= ties_left)
            m = (gt | take) & (kv > 0)
            plsc.store_scatter(
                row_v, [iv], lax.bitcast_convert_type(kv, jnp.float32), mask=m)
            return ties_left - jnp.sum(eq.astype(jnp.int32))

        lax.fori_loop(0, nv, sc_body, k1 - cnt_gt)

        pltpu.sync_copy(row_v, o_hbm.at[pl.ds(base, N)])


@jax.jit
def _topk_sc(x_flat):
    mesh = plsc.VectorSubcoreMesh(core_axis_name="c", subcore_axis_name="s")
    f = pl.kernel(
        _topk_body,
        out_type=jax.ShapeDtypeStruct((ROWS * N,), jnp.float32),
        mesh=mesh,
        scratch_types=[
            pltpu.VMEM((N,), jnp.float32),
            pltpu.VMEM((4096,), jnp.int32),
            pltpu.VMEM((CAP,), jnp.int32),
            pltpu.VMEM((CAP,), jnp.int32),
        ],
        compiler_params=pltpu.CompilerParams(needs_layout_passes=False),
    )
    return f(x_flat)


def kernel(x):
    out = _topk_sc(x.reshape(-1))
    return out.reshape(ROWS, N)


# speedup vs baseline: 8.9319x; 3.4500x over previous
import jax
import jax.numpy as jnp
from jax import lax
from jax.experimental import pallas as pl
from jax.experimental.pallas import tpu as pltpu
from jax.experimental.pallas import tpu_sc as plsc

K = 64
ROWS = 128
N = 32768
L = 16
NV = N // L
NWORKERS = 32
RPW = ROWS // NWORKERS
CAP = 2048
NEG_INF_KEY = -(2 ** 31)
ZCHUNK = 8192
NZ = N // ZCHUNK


def _scalar(v16):
    return jnp.squeeze(lax.slice(v16, (0,), (1,)))


def _topk_body(x_hbm, o_hbm, rowbuf0, rowbuf1, hist_v, ckey_v, cidx_v,
               wstage_val, wstage_idx, wval_v, wgidx_v, zero_v,
               sem_in0, sem_in1, sem_z, sem_s):
    wid = lax.axis_index("s") * 2 + lax.axis_index("c")
    iota = lax.iota(jnp.int32, L)
    ones = jnp.ones((L,), jnp.int32)

    sems_in = [sem_in0, sem_in1]
    rowbufs = [rowbuf0, rowbuf1]

    @plsc.parallel_loop(0, ZCHUNK // L, unroll=8)
    def _(j):
        zero_v[pl.ds(j * L, L)] = jnp.zeros((L,), jnp.float32)

    row0 = wid * RPW
    pltpu.async_copy(x_hbm.at[pl.ds(row0 * N, N)], rowbuf0, sem_in0)

    for r in range(RPW):
        row = row0 + r
        base = row * N
        row_v = rowbufs[r % 2]

        zdmas = [
            pltpu.async_copy(
                zero_v, o_hbm.at[pl.ds(base + q * ZCHUNK, ZCHUNK)], sem_z)
            for q in range(NZ)
        ]
        if r + 1 < RPW:
            pltpu.async_copy(
                x_hbm.at[pl.ds((row + 1) * N, N)],
                rowbufs[(r + 1) % 2],
                sems_in[(r + 1) % 2],
            )

        pltpu.make_async_copy(
            x_hbm.at[pl.ds(base, N)], row_v, sems_in[r % 2]).wait()

        @plsc.parallel_loop(0, 4096 // L, unroll=8)
        def _(j):
            hist_v[pl.ds(j * L, L)] = jnp.zeros((L,), jnp.int32)

        @plsc.parallel_loop(0, NV, unroll=8)
        def _(i):
            v = row_v[pl.ds(i * L, L)]
            u = lax.bitcast_convert_type(v, jnp.int32)
            key = u ^ (lax.shift_right_arithmetic(u, 31) & 0x7FFFFFFF)
            b = lax.shift_right_arithmetic(key, 20) + 2048
            plsc.addupdate_scatter(hist_v, [b], ones)

        def scan_cond(c):
            return jnp.logical_not(c[2])

        def scan_body(c):
            j, cum, found, b1 = c
            hv = hist_v[pl.ds(j * L, L)]
            s = jnp.sum(hv)
            found_here = (cum + s) >= K
            pref = plsc.cumsum(hv)
            suf_in = s - pref + hv
            cross = (cum + suf_in) >= K
            npos = jnp.sum(cross.astype(jnp.int32))
            b1_here = j * L + npos - 1
            b1_n = jnp.where(found_here, b1_here, b1)
            cum_n = jnp.where(found_here, cum, cum + s)
            return (j - 1, cum_n, found | found_here, b1_n)

        init = (jnp.int32(4096 // L - 1), jnp.int32(0), False, jnp.int32(0))
        _, _, _, b1 = lax.while_loop(scan_cond, scan_body, init)

        lo_edge = lax.shift_left(b1 - 2048, 20)

        @plsc.parallel_loop(0, NV, unroll=4, carry=jnp.int32(0))
        def coff_final(i, coff):
            v = row_v[pl.ds(i * L, L)]
            u = lax.bitcast_convert_type(v, jnp.int32)
            key = u ^ (lax.shift_right_arithmetic(u, 31) & 0x7FFFFFFF)
            in_b = key >= lo_edge
            plsc.store_compressed(ckey_v.at[pl.ds(coff, L)], key, mask=in_b)
            plsc.store_compressed(
                cidx_v.at[pl.ds(coff, L)], iota + i * L, mask=in_b)
            cnt = _scalar(plsc.all_reduce_population_count(in_b))
            return jnp.minimum(coff + cnt, CAP - L)

        c = coff_final

        ckey_v[pl.ds(c, L)] = jnp.full((L,), NEG_INF_KEY, jnp.int32)
        nv = (c + L - 1) // L

        def bs_body(it, t):
            cand = t + lax.shift_left(1, 19 - it)

            @plsc.parallel_loop(0, nv, unroll=4,
                                carry=jnp.zeros((L,), jnp.int32))
            def acc_final(j, acc):
                kv = ckey_v[pl.ds(j * L, L)]
                return acc + plsc.all_reduce_population_count(kv >= cand)

            cnt = _scalar(acc_final)
            return jnp.where(cnt >= K, cand, t)

        t = lax.fori_loop(0, 20, bs_body, lo_edge)

        @plsc.parallel_loop(0, nv, unroll=4, carry=jnp.zeros((L,), jnp.int32))
        def gt_final(j, acc):
            kv = ckey_v[pl.ds(j * L, L)]
            return acc + plsc.all_reduce_population_count(kv > t)

        cnt_gt = _scalar(gt_final)

        if r > 0:
            pltpu.make_async_copy(
                wval_v, o_hbm.at[wgidx_v], sem_s).wait()

        def win_body(j, carry):
            ties_left, woff = carry
            kv = ckey_v[pl.ds(j * L, L)]
            iv = cidx_v[pl.ds(j * L, L)]
            gt = kv > t
            eq = kv == t
            pr = plsc.cumsum(eq.astype(jnp.int32))
            take = eq & (pr <= ties_left)
            m = gt | take
            wv = lax.bitcast_convert_type(
                jnp.maximum(kv, 0), jnp.float32)
            plsc.store_compressed(
                wstage_val.at[pl.ds(woff, L)], wv, mask=m)
            plsc.store_compressed(
                wstage_idx.at[pl.ds(woff, L)], iv + base, mask=m)
            ties_left -= _scalar(plsc.all_reduce_population_count(eq))
            woff += _scalar(plsc.all_reduce_population_count(m))
            return (ties_left, woff)

        lax.fori_loop(0, nv, win_body, (K - cnt_gt, jnp.int32(0)))

        for j in range(K // L):
            wval_v[pl.ds(j * L, L)] = wstage_val[pl.ds(j * L, L)]
            wgidx_v[pl.ds(j * L, L)] = wstage_idx[pl.ds(j * L, L)]

        for d in zdmas:
            d.wait()
        pltpu.async_copy(wval_v, o_hbm.at[wgidx_v], sem_s)

    pltpu.make_async_copy(wval_v, o_hbm.at[wgidx_v], sem_s).wait()


@jax.jit
def _topk_sc(x_flat):
    mesh = plsc.VectorSubcoreMesh(core_axis_name="c", subcore_axis_name="s")
    f = pl.kernel(
        _topk_body,
        out_type=jax.ShapeDtypeStruct((ROWS * N,), jnp.float32),
        mesh=mesh,
        scratch_types=[
            pltpu.VMEM((N,), jnp.float32),
            pltpu.VMEM((N,), jnp.float32),
            pltpu.VMEM((4096,), jnp.int32),
            pltpu.VMEM((CAP,), jnp.int32),
            pltpu.VMEM((CAP,), jnp.int32),
            pltpu.VMEM((K + L,), jnp.float32),
            pltpu.VMEM((K + L,), jnp.int32),
            pltpu.VMEM((K,), jnp.float32),
            pltpu.VMEM((K,), jnp.int32),
            pltpu.VMEM((ZCHUNK,), jnp.float32),
            pltpu.SemaphoreType.DMA,
            pltpu.SemaphoreType.DMA,
            pltpu.SemaphoreType.DMA,
            pltpu.SemaphoreType.DMA,
        ],
        compiler_params=pltpu.CompilerParams(needs_layout_passes=False),
    )
    return f(x_flat)


def kernel(x):
    out = _topk_sc(x.reshape(-1))
    return out.reshape(ROWS, N)
